# baseline (speedup 1.0000x reference)
import functools

import jax
import jax.numpy as jnp
from jax import lax
from jax.experimental import pallas as pl
from jax.experimental.pallas import tpu as pltpu
from jax.experimental.pallas import tpu_sc as plsc

_BATCH = 16384
_DIM = 32
_NUM_WORKERS = 32
_BPW = _BATCH // _NUM_WORKERS
_LANES = 16


def _gmf_body(users_hbm, items_hbm, ut_hbm, it_hbm, out_hbm,
              idx_u, idx_i, rows_u, rows_i, sem_u, sem_i):
    wid = lax.axis_index("s") * 2 + lax.axis_index("c")
    base = wid * _BPW
    pltpu.sync_copy(users_hbm.at[pl.ds(base, _BPW)], idx_u)
    pltpu.sync_copy(items_hbm.at[pl.ds(base, _BPW)], idx_i)
    cp_u = pltpu.async_copy(ut_hbm.at[idx_u], rows_u, sem_u)
    cp_i = pltpu.async_copy(it_hbm.at[idx_i], rows_i, sem_i)
    cp_u.wait()
    cp_i.wait()

    def body(j, carry):
        for h in range(_DIM // _LANES):
            sl = pl.ds(h * _LANES, _LANES)
            rows_u[j, sl] = rows_u[j, sl] * rows_i[j, sl]
        return carry

    lax.fori_loop(0, _BPW, body, 0, unroll=4)
    pltpu.sync_copy(rows_u, out_hbm.at[pl.ds(base, _BPW)])


@jax.jit
def kernel(users, items, user_table, item_table):
    mesh = plsc.VectorSubcoreMesh(core_axis_name="c", subcore_axis_name="s")
    run = functools.partial(
        pl.kernel,
        mesh=mesh,
        compiler_params=pltpu.CompilerParams(use_tc_tiling_on_sc=False),
        out_type=jax.ShapeDtypeStruct((_BATCH, _DIM), jnp.float32),
        scratch_types=[
            pltpu.VMEM((_BPW,), jnp.int32),
            pltpu.VMEM((_BPW,), jnp.int32),
            pltpu.VMEM((_BPW, _DIM), jnp.float32),
            pltpu.VMEM((_BPW, _DIM), jnp.float32),
            pltpu.SemaphoreType.DMA,
            pltpu.SemaphoreType.DMA,
        ],
    )(_gmf_body)
    return run(users.astype(jnp.int32), items.astype(jnp.int32),
               user_table, item_table)
